# baseline (speedup 1.0000x reference)
import functools
import jax
import jax.numpy as jnp
from jax import lax
from jax.experimental import pallas as pl
from jax.experimental.pallas import tpu as pltpu
from jax.experimental.pallas import tpu_sc as plsc

N = 10000
E = 320000
IN_DIM = 128
OUT_DIM = 100

NP = 10240
DP = 112
NC = 2
NS = 16
NW = NC * NS
CW = 128
CH = 80
EPW = CH * CW
EPAD = NW * EPW
RPT = NP // NS

@functools.cache
def _sc_kernels():
    mesh = plsc.VectorSubcoreMesh(
        core_axis_name="c", subcore_axis_name="s",
        num_cores=NC, num_subcores=NS)

    params = pltpu.CompilerParams(use_tc_tiling_on_sc=False)

    deg_kernel = functools.partial(
        pl.kernel,
        out_type=jax.ShapeDtypeStruct((NC, NP, 16), jnp.float32),
        mesh=mesh,
        compiler_params=params,
        scratch_types=[
            pltpu.VMEM((CH, CW), jnp.int32),
            pltpu.VMEM((CW, 16), jnp.float32),
            pltpu.VMEM_SHARED((NP, 16), jnp.float32),
            pltpu.SemaphoreType.DMA,
        ],
    )(_deg_body)

    edge_kernel = functools.partial(
        pl.kernel,
        out_type=jax.ShapeDtypeStruct((NC, NP, DP), jnp.float32),
        mesh=mesh,
        compiler_params=params,
        scratch_types=[
            pltpu.VMEM((CH, CW), jnp.int32),
            pltpu.VMEM((CH, CW), jnp.int32),
            pltpu.VMEM((CW, DP), jnp.float32),
            pltpu.VMEM((CW, DP), jnp.float32),
            pltpu.VMEM_SHARED((NP, DP), jnp.float32),
            pltpu.SemaphoreType.DMA,
            pltpu.SemaphoreType.DMA,
        ],
    )(_edge_body)

    return deg_kernel, edge_kernel


def _deg_body(dst_hbm, zeros_hbm, ones_hbm, deg_out, dstv, onesv, acc, sem):
    c = lax.axis_index("c")
    s = lax.axis_index("s")
    wid = s * NC + c
    pltpu.sync_copy(zeros_hbm, acc.at[pl.ds(s * RPT, RPT)])
    pltpu.sync_copy(dst_hbm.at[wid], dstv)
    pltpu.sync_copy(ones_hbm, onesv)
    plsc.subcore_barrier()

    def body(j, carry):
        pltpu.sync_copy(onesv, acc.at[dstv.at[j]], add=True)
        return carry

    lax.fori_loop(0, CH, body, 0)
    plsc.subcore_barrier()
    pltpu.sync_copy(acc.at[pl.ds(s * RPT, RPT)],
                    deg_out.at[c, pl.ds(s * RPT, RPT)])


def _edge_body(hp_hbm, src_hbm, dst_hbm, zeros_hbm, acc_out,
               srcv, dstv, rows_a, rows_b, acc, sem_a, sem_b):
    c = lax.axis_index("c")
    s = lax.axis_index("s")
    wid = s * NC + c
    pltpu.sync_copy(zeros_hbm, acc.at[pl.ds(s * RPT, RPT)])
    pltpu.sync_copy(src_hbm.at[wid], srcv)
    pltpu.sync_copy(dst_hbm.at[wid], dstv)
    plsc.subcore_barrier()

    @pl.when(c == 0)
    def _pipelined():
        pltpu.async_copy(hp_hbm.at[srcv.at[0]], rows_a, sem_a)

        def body(j2, carry):
            j = j2 * 2
            pltpu.make_async_copy(hp_hbm.at[srcv.at[j]], rows_a, sem_a).wait()
            pltpu.async_copy(hp_hbm.at[srcv.at[j + 1]], rows_b, sem_b)
            pltpu.sync_copy(rows_a, acc.at[dstv.at[j]], add=True)
            pltpu.make_async_copy(
                hp_hbm.at[srcv.at[j + 1]], rows_b, sem_b).wait()
            nxt = jnp.minimum(j + 2, CH - 1)
            pltpu.async_copy(hp_hbm.at[srcv.at[nxt]], rows_a, sem_a)
            pltpu.sync_copy(rows_b, acc.at[dstv.at[j + 1]], add=True)
            return carry

        lax.fori_loop(0, CH // 2, body, 0)
        pltpu.make_async_copy(hp_hbm.at[srcv.at[CH - 1]], rows_a, sem_a).wait()

    @pl.when(c != 0)
    def _serial():
        def body(j, carry):
            pltpu.async_copy(hp_hbm.at[srcv.at[j]], rows_a, sem_a).wait()
            pltpu.sync_copy(rows_a, acc.at[dstv.at[j]], add=True)
            return carry

        lax.fori_loop(0, CH, body, 0)

    plsc.subcore_barrier()
    pltpu.sync_copy(acc.at[pl.ds(s * RPT, RPT)],
                    acc_out.at[c, pl.ds(s * RPT, RPT)])


_BLK = 1024


def _scale_mm_body(x_ref, wt_ref, da_ref, db_ref, hp_ref, dis_ref):
    deg = da_ref[:, 0:1] + db_ref[:, 0:1] + 1.0
    dis = jnp.broadcast_to(lax.rsqrt(deg), (_BLK, DP))
    h = jnp.dot(x_ref[...], wt_ref[...], preferred_element_type=jnp.float32)
    hp_ref[...] = dis * h
    dis_ref[...] = dis


_scale_mm = pl.pallas_call(
    _scale_mm_body,
    grid=(NP // _BLK,),
    in_specs=[
        pl.BlockSpec((_BLK, IN_DIM), lambda i: (i, 0)),
        pl.BlockSpec((IN_DIM, DP), lambda i: (0, 0)),
        pl.BlockSpec((_BLK, 16), lambda i: (i, 0)),
        pl.BlockSpec((_BLK, 16), lambda i: (i, 0)),
    ],
    out_specs=[
        pl.BlockSpec((_BLK, DP), lambda i: (i, 0)),
        pl.BlockSpec((_BLK, DP), lambda i: (i, 0)),
    ],
    out_shape=[
        jax.ShapeDtypeStruct((NP, DP), jnp.float32),
        jax.ShapeDtypeStruct((NP, DP), jnp.float32),
    ],
)


def _epilogue_body(a0_ref, a1_ref, hp_ref, dis_ref, b_ref, out_ref):
    agg = a0_ref[...] + a1_ref[...] + hp_ref[...]
    out_ref[...] = dis_ref[...] * agg + b_ref[0:1, :]


_epilogue = pl.pallas_call(
    _epilogue_body,
    grid=(NP // _BLK,),
    in_specs=[
        pl.BlockSpec((_BLK, DP), lambda i: (i, 0)),
        pl.BlockSpec((_BLK, DP), lambda i: (i, 0)),
        pl.BlockSpec((_BLK, DP), lambda i: (i, 0)),
        pl.BlockSpec((_BLK, DP), lambda i: (i, 0)),
        pl.BlockSpec((8, DP), lambda i: (0, 0)),
    ],
    out_specs=pl.BlockSpec((_BLK, DP), lambda i: (i, 0)),
    out_shape=jax.ShapeDtypeStruct((NP, DP), jnp.float32),
)


@jax.jit
def kernel(x, edge_index, W, b):
    src = edge_index[0].astype(jnp.int32)
    dst = edge_index[1].astype(jnp.int32)
    padv = jnp.full((EPAD - E,), N, dtype=jnp.int32)
    src_p = jnp.concatenate([src, padv]).reshape(NW, CH, CW)
    dst_p = jnp.concatenate([dst, padv]).reshape(NW, CH, CW)

    x_p = jnp.pad(x, ((0, NP - N), (0, 0)))
    wt_p = jnp.pad(W.T, ((0, 0), (0, DP - OUT_DIM)))
    b_p = jnp.broadcast_to(jnp.pad(b, (0, DP - OUT_DIM))[None, :], (8, DP))

    zeros16 = jnp.zeros((RPT, 16), jnp.float32)
    ones16 = jnp.ones((CW, 16), jnp.float32)
    zerosDP = jnp.zeros((RPT, DP), jnp.float32)

    deg_kernel, edge_kernel = _sc_kernels()
    deg2 = deg_kernel(dst_p, zeros16, ones16)
    hp, dis = _scale_mm(x_p, wt_p, deg2[0], deg2[1])
    acc2 = edge_kernel(hp, src_p, dst_p, zerosDP)
    out = _epilogue(acc2[0], acc2[1], hp, dis, b_p)
    return out[:N, :OUT_DIM]


# speedup vs baseline: 2.7647x; 2.7647x over previous
import functools
import jax
import jax.numpy as jnp
from jax import lax
from jax.experimental import pallas as pl
from jax.experimental.pallas import tpu as pltpu
from jax.experimental.pallas import tpu_sc as plsc

N = 10000
E = 320000
IN_DIM = 128
OUT_DIM = 100

NP = 10240
DP = 112
NC = 2
NS = 16
NW = NC * NS
CW = 128
CH = 80
EPW = CH * CW
EPAD = NW * EPW
RPT = NP // NS

@functools.cache
def _sc_kernels():
    mesh = plsc.VectorSubcoreMesh(
        core_axis_name="c", subcore_axis_name="s",
        num_cores=NC, num_subcores=NS)

    params = pltpu.CompilerParams(use_tc_tiling_on_sc=False)

    deg_kernel = functools.partial(
        pl.kernel,
        out_type=jax.ShapeDtypeStruct((NC, NP, 16), jnp.float32),
        mesh=mesh,
        compiler_params=params,
        scratch_types=[
            pltpu.VMEM((CH, CW), jnp.int32),
            pltpu.VMEM((CW, 16), jnp.float32),
            pltpu.VMEM_SHARED((NP, 16), jnp.float32),
            pltpu.SemaphoreType.DMA,
        ],
    )(_deg_body)

    edge_kernel = functools.partial(
        pl.kernel,
        out_type=jax.ShapeDtypeStruct((NC, NP, DP), jnp.float32),
        mesh=mesh,
        compiler_params=params,
        scratch_types=[
            pltpu.VMEM((CH, CW), jnp.int32),
            pltpu.VMEM((CH, CW), jnp.int32),
            pltpu.VMEM((CW, DP), jnp.float32),
            pltpu.VMEM((CW, DP), jnp.float32),
            pltpu.VMEM_SHARED((NP, DP), jnp.float32),
            pltpu.SemaphoreType.DMA,
            pltpu.SemaphoreType.DMA,
        ],
    )(_edge_body)

    return deg_kernel, edge_kernel


def _deg_body(dst_hbm, zeros_hbm, ones_hbm, deg_out, dstv, onesv, acc, sem):
    c = lax.axis_index("c")
    s = lax.axis_index("s")
    wid = s * NC + c
    pltpu.sync_copy(zeros_hbm, acc.at[pl.ds(s * RPT, RPT)])
    pltpu.sync_copy(dst_hbm.at[wid], dstv)
    pltpu.sync_copy(ones_hbm, onesv)
    plsc.subcore_barrier()

    def body(j, carry):
        pltpu.sync_copy(onesv, acc.at[dstv.at[j]], add=True)
        return carry

    lax.fori_loop(0, CH, body, 0)
    plsc.subcore_barrier()
    pltpu.sync_copy(acc.at[pl.ds(s * RPT, RPT)],
                    deg_out.at[c, pl.ds(s * RPT, RPT)])


def _edge_body(hp_hbm, src_hbm, dst_hbm, zeros_hbm, acc_out,
               srcv, dstv, rows_a, rows_b, acc, sem_a, sem_b):
    c = lax.axis_index("c")
    s = lax.axis_index("s")
    wid = s * NC + c
    pltpu.sync_copy(zeros_hbm, acc.at[pl.ds(s * RPT, RPT)])
    pltpu.sync_copy(src_hbm.at[wid], srcv)
    pltpu.sync_copy(dst_hbm.at[wid], dstv)
    plsc.subcore_barrier()

    pltpu.async_copy(hp_hbm.at[srcv.at[0]], rows_a, sem_a)

    def body(j2, carry):
        j = j2 * 2
        pltpu.make_async_copy(hp_hbm.at[srcv.at[j]], rows_a, sem_a).wait()
        pltpu.async_copy(hp_hbm.at[srcv.at[j + 1]], rows_b, sem_b)
        pltpu.sync_copy(rows_a, acc.at[dstv.at[j]], add=True)
        pltpu.make_async_copy(hp_hbm.at[srcv.at[j + 1]], rows_b, sem_b).wait()
        nxt = jnp.minimum(j + 2, CH - 1)
        pltpu.async_copy(hp_hbm.at[srcv.at[nxt]], rows_a, sem_a)
        pltpu.sync_copy(rows_b, acc.at[dstv.at[j + 1]], add=True)
        return carry

    lax.fori_loop(0, CH // 2, body, 0)
    pltpu.make_async_copy(hp_hbm.at[srcv.at[CH - 1]], rows_a, sem_a).wait()
    plsc.subcore_barrier()
    pltpu.sync_copy(acc.at[pl.ds(s * RPT, RPT)],
                    acc_out.at[c, pl.ds(s * RPT, RPT)])


_BLK = 1024


def _scale_mm_body(x_ref, wt_ref, da_ref, db_ref, hp_ref, dis_ref):
    deg = da_ref[:, 0:1] + db_ref[:, 0:1] + 1.0
    dis = jnp.broadcast_to(lax.rsqrt(deg), (_BLK, DP))
    h = jnp.dot(x_ref[...], wt_ref[...], preferred_element_type=jnp.float32)
    hp_ref[...] = dis * h
    dis_ref[...] = dis


_scale_mm = pl.pallas_call(
    _scale_mm_body,
    grid=(NP // _BLK,),
    in_specs=[
        pl.BlockSpec((_BLK, IN_DIM), lambda i: (i, 0)),
        pl.BlockSpec((IN_DIM, DP), lambda i: (0, 0)),
        pl.BlockSpec((_BLK, 16), lambda i: (i, 0)),
        pl.BlockSpec((_BLK, 16), lambda i: (i, 0)),
    ],
    out_specs=[
        pl.BlockSpec((_BLK, DP), lambda i: (i, 0)),
        pl.BlockSpec((_BLK, DP), lambda i: (i, 0)),
    ],
    out_shape=[
        jax.ShapeDtypeStruct((NP, DP), jnp.float32),
        jax.ShapeDtypeStruct((NP, DP), jnp.float32),
    ],
)


def _epilogue_body(a0_ref, a1_ref, hp_ref, dis_ref, b_ref, out_ref):
    agg = a0_ref[...] + a1_ref[...] + hp_ref[...]
    out_ref[...] = dis_ref[...] * agg + b_ref[0:1, :]


_epilogue = pl.pallas_call(
    _epilogue_body,
    grid=(NP // _BLK,),
    in_specs=[
        pl.BlockSpec((_BLK, DP), lambda i: (i, 0)),
        pl.BlockSpec((_BLK, DP), lambda i: (i, 0)),
        pl.BlockSpec((_BLK, DP), lambda i: (i, 0)),
        pl.BlockSpec((_BLK, DP), lambda i: (i, 0)),
        pl.BlockSpec((8, DP), lambda i: (0, 0)),
    ],
    out_specs=pl.BlockSpec((_BLK, DP), lambda i: (i, 0)),
    out_shape=jax.ShapeDtypeStruct((NP, DP), jnp.float32),
)


@jax.jit
def kernel(x, edge_index, W, b):
    src = edge_index[0].astype(jnp.int32)
    dst = edge_index[1].astype(jnp.int32)
    padv = N + (jnp.arange(EPAD - E, dtype=jnp.int32) % (NP - N))
    src_p = jnp.concatenate([src, padv]).reshape(NW, CH, CW)
    dst_p = jnp.concatenate([dst, padv]).reshape(NW, CH, CW)

    x_p = jnp.pad(x, ((0, NP - N), (0, 0)))
    wt_p = jnp.pad(W.T, ((0, 0), (0, DP - OUT_DIM)))
    b_p = jnp.broadcast_to(jnp.pad(b, (0, DP - OUT_DIM))[None, :], (8, DP))

    zeros16 = jnp.zeros((RPT, 16), jnp.float32)
    ones16 = jnp.ones((CW, 16), jnp.float32)
    zerosDP = jnp.zeros((RPT, DP), jnp.float32)

    deg_kernel, edge_kernel = _sc_kernels()
    deg2 = deg_kernel(dst_p, zeros16, ones16)
    hp, dis = _scale_mm(x_p, wt_p, deg2[0], deg2[1])
    acc2 = edge_kernel(hp, src_p, dst_p, zerosDP)
    out = _epilogue(acc2[0], acc2[1], hp, dis, b_p)
    return out[:N, :OUT_DIM]


# speedup vs baseline: 2.9755x; 1.0763x over previous
import functools
import jax
import jax.numpy as jnp
from jax import lax
from jax.experimental import pallas as pl
from jax.experimental.pallas import tpu as pltpu
from jax.experimental.pallas import tpu_sc as plsc

N = 10000
E = 320000
IN_DIM = 128
OUT_DIM = 100

DP = 112
NC = 2
NS = 16
NW = NC * NS
CW = 128
ROWS = E // CW
CPW = ROWS // NW
XW = ROWS - NW * CPW
RPT = N // NS


@functools.cache
def _sc_kernels():
    mesh = plsc.VectorSubcoreMesh(
        core_axis_name="c", subcore_axis_name="s",
        num_cores=NC, num_subcores=NS)

    params = pltpu.CompilerParams(use_tc_tiling_on_sc=False)

    deg_kernel = functools.partial(
        pl.kernel,
        out_type=jax.ShapeDtypeStruct((NC, N, 16), jnp.float32),
        mesh=mesh,
        compiler_params=params,
        scratch_types=[
            pltpu.VMEM((CPW + 1, CW), jnp.int32),
            pltpu.VMEM((CW, 16), jnp.float32),
            pltpu.VMEM_SHARED((N, 16), jnp.float32),
        ],
    )(_deg_body)

    edge_kernel = functools.partial(
        pl.kernel,
        out_type=jax.ShapeDtypeStruct((NC, N, DP), jnp.float32),
        mesh=mesh,
        compiler_params=params,
        scratch_types=[
            pltpu.VMEM((CPW + 1, CW), jnp.int32),
            pltpu.VMEM((CPW + 1, CW), jnp.int32),
            pltpu.VMEM((CW, DP), jnp.float32),
            pltpu.VMEM((CW, DP), jnp.float32),
            pltpu.VMEM_SHARED((N, DP), jnp.float32),
            pltpu.SemaphoreType.DMA,
            pltpu.SemaphoreType.DMA,
        ],
    )(_edge_body)

    return deg_kernel, edge_kernel


def _deg_body(dst_hbm, zeros_hbm, ones_hbm, deg_out, dstv, onesv, acc):
    c = lax.axis_index("c")
    s = lax.axis_index("s")
    wid = s * NC + c
    pltpu.sync_copy(zeros_hbm, acc.at[pl.ds(s * RPT, RPT)])
    pltpu.sync_copy(dst_hbm.at[pl.ds(wid * CPW, CPW)],
                    dstv.at[pl.ds(0, CPW)])
    pltpu.sync_copy(ones_hbm, onesv)

    @pl.when(wid < XW)
    def _():
        pltpu.sync_copy(dst_hbm.at[pl.ds(NW * CPW + wid, 1)],
                        dstv.at[pl.ds(CPW, 1)])

    plsc.subcore_barrier()

    def body(j, carry):
        pltpu.sync_copy(onesv, acc.at[dstv.at[j]], add=True)
        return carry

    lax.fori_loop(0, CPW, body, 0)

    @pl.when(wid < XW)
    def _():
        pltpu.sync_copy(onesv, acc.at[dstv.at[CPW]], add=True)

    plsc.subcore_barrier()
    pltpu.sync_copy(acc.at[pl.ds(s * RPT, RPT)],
                    deg_out.at[c, pl.ds(s * RPT, RPT)])


def _edge_body(hp_hbm, src_hbm, dst_hbm, zeros_hbm, acc_out,
               srcv, dstv, rows_a, rows_b, acc, sem_a, sem_b):
    c = lax.axis_index("c")
    s = lax.axis_index("s")
    wid = s * NC + c
    pltpu.sync_copy(zeros_hbm, acc.at[pl.ds(s * RPT, RPT)])
    pltpu.sync_copy(src_hbm.at[pl.ds(wid * CPW, CPW)],
                    srcv.at[pl.ds(0, CPW)])
    pltpu.sync_copy(dst_hbm.at[pl.ds(wid * CPW, CPW)],
                    dstv.at[pl.ds(0, CPW)])

    @pl.when(wid < XW)
    def _():
        pltpu.sync_copy(src_hbm.at[pl.ds(NW * CPW + wid, 1)],
                        srcv.at[pl.ds(CPW, 1)])
        pltpu.sync_copy(dst_hbm.at[pl.ds(NW * CPW + wid, 1)],
                        dstv.at[pl.ds(CPW, 1)])

    plsc.subcore_barrier()

    pltpu.async_copy(hp_hbm.at[srcv.at[0]], rows_a, sem_a)

    def body(j2, carry):
        j = j2 * 2
        pltpu.make_async_copy(hp_hbm.at[srcv.at[j]], rows_a, sem_a).wait()
        pltpu.async_copy(hp_hbm.at[srcv.at[j + 1]], rows_b, sem_b)
        pltpu.sync_copy(rows_a, acc.at[dstv.at[j]], add=True)
        pltpu.make_async_copy(hp_hbm.at[srcv.at[j + 1]], rows_b, sem_b).wait()
        nxt = jnp.minimum(j + 2, CPW - 1)
        pltpu.async_copy(hp_hbm.at[srcv.at[nxt]], rows_a, sem_a)
        pltpu.sync_copy(rows_b, acc.at[dstv.at[j + 1]], add=True)
        return carry

    lax.fori_loop(0, CPW // 2, body, 0)
    pltpu.make_async_copy(hp_hbm.at[srcv.at[CPW - 1]], rows_a, sem_a).wait()

    @pl.when(wid < XW)
    def _():
        pltpu.async_copy(hp_hbm.at[srcv.at[CPW]], rows_a, sem_a).wait()
        pltpu.sync_copy(rows_a, acc.at[dstv.at[CPW]], add=True)

    plsc.subcore_barrier()
    pltpu.sync_copy(acc.at[pl.ds(s * RPT, RPT)],
                    acc_out.at[c, pl.ds(s * RPT, RPT)])


_BLK = 1000


def _scale_mm_body(x_ref, wt_ref, deg_ref, hp_ref):
    deg = deg_ref[0, :, 0:1] + deg_ref[1, :, 0:1] + 1.0
    dis = jnp.broadcast_to(lax.rsqrt(deg), (_BLK, DP))
    h = jnp.dot(x_ref[...], wt_ref[...], preferred_element_type=jnp.float32)
    hp_ref[...] = dis * h


_scale_mm = pl.pallas_call(
    _scale_mm_body,
    grid=(N // _BLK,),
    in_specs=[
        pl.BlockSpec((_BLK, IN_DIM), lambda i: (i, 0)),
        pl.BlockSpec((IN_DIM, DP), lambda i: (0, 0)),
        pl.BlockSpec((NC, _BLK, 16), lambda i: (0, i, 0)),
    ],
    out_specs=pl.BlockSpec((_BLK, DP), lambda i: (i, 0)),
    out_shape=jax.ShapeDtypeStruct((N, DP), jnp.float32),
)


def _epilogue_body(acc_ref, hp_ref, deg_ref, b_ref, out_ref):
    deg = deg_ref[0, :, 0:1] + deg_ref[1, :, 0:1] + 1.0
    dis = lax.rsqrt(deg)
    agg = acc_ref[0] + acc_ref[1] + hp_ref[...]
    out_ref[...] = (dis * agg)[:, :OUT_DIM] + b_ref[0:1, :OUT_DIM]


_epilogue = pl.pallas_call(
    _epilogue_body,
    grid=(N // _BLK,),
    in_specs=[
        pl.BlockSpec((NC, _BLK, DP), lambda i: (0, i, 0)),
        pl.BlockSpec((_BLK, DP), lambda i: (i, 0)),
        pl.BlockSpec((NC, _BLK, 16), lambda i: (0, i, 0)),
        pl.BlockSpec((8, DP), lambda i: (0, 0)),
    ],
    out_specs=pl.BlockSpec((_BLK, OUT_DIM), lambda i: (i, 0)),
    out_shape=jax.ShapeDtypeStruct((N, OUT_DIM), jnp.float32),
)


@jax.jit
def kernel(x, edge_index, W, b):
    src2 = edge_index[0].astype(jnp.int32).reshape(ROWS, CW)
    dst2 = edge_index[1].astype(jnp.int32).reshape(ROWS, CW)

    wt_p = jnp.pad(W.T, ((0, 0), (0, DP - OUT_DIM)))
    b_p = jnp.broadcast_to(jnp.pad(b, (0, DP - OUT_DIM))[None, :], (8, DP))

    zeros16 = jnp.zeros((RPT, 16), jnp.float32)
    ones16 = jnp.ones((CW, 16), jnp.float32)
    zerosDP = jnp.zeros((RPT, DP), jnp.float32)

    deg_kernel, edge_kernel = _sc_kernels()
    deg2 = deg_kernel(dst2, zeros16, ones16)
    hp = _scale_mm(x, wt_p, deg2)
    acc2 = edge_kernel(hp, src2, dst2, zerosDP)
    return _epilogue(acc2, hp, deg2, b_p)


# speedup vs baseline: 3.0605x; 1.0286x over previous
import functools
import jax
import jax.numpy as jnp
from jax import lax
from jax.experimental import pallas as pl
from jax.experimental.pallas import tpu as pltpu
from jax.experimental.pallas import tpu_sc as plsc

N = 10000
E = 320000
IN_DIM = 128
OUT_DIM = 100

DP = 112
NC = 2
NS = 16
NW = NC * NS
CW = 128
ROWS = E // CW
CPW = ROWS // NW
XW = ROWS - NW * CPW
RPT = N // NS


@functools.cache
def _sc_kernels():
    mesh = plsc.VectorSubcoreMesh(
        core_axis_name="c", subcore_axis_name="s",
        num_cores=NC, num_subcores=NS)

    params = pltpu.CompilerParams(use_tc_tiling_on_sc=False)

    deg_kernel = functools.partial(
        pl.kernel,
        out_type=jax.ShapeDtypeStruct((NC, N, 16), jnp.float32),
        mesh=mesh,
        compiler_params=params,
        scratch_types=[
            pltpu.VMEM((CPW + 1, CW), jnp.int32),
            pltpu.VMEM((CW, 16), jnp.float32),
            pltpu.VMEM_SHARED((N, 16), jnp.float32),
            pltpu.SemaphoreType.DMA,
        ],
    )(_deg_body)

    edge_kernel = functools.partial(
        pl.kernel,
        out_type=jax.ShapeDtypeStruct((NC, N, DP), jnp.float32),
        mesh=mesh,
        compiler_params=params,
        scratch_types=[
            pltpu.VMEM((CPW + 1, CW), jnp.int32),
            pltpu.VMEM((CPW + 1, CW), jnp.int32),
            pltpu.VMEM((CW, DP), jnp.float32),
            pltpu.VMEM((CW, DP), jnp.float32),
            pltpu.VMEM_SHARED((N, DP), jnp.float32),
            pltpu.SemaphoreType.DMA,
            pltpu.SemaphoreType.DMA,
        ],
    )(_edge_body)

    return deg_kernel, edge_kernel


def _deg_body(dst_hbm, zeros_hbm, ones_hbm, deg_out, dstv, onesv, acc, sem):
    c = lax.axis_index("c")
    s = lax.axis_index("s")
    wid = s * NC + c
    pltpu.sync_copy(zeros_hbm, acc.at[pl.ds(s * RPT, RPT)])
    pltpu.sync_copy(dst_hbm.at[pl.ds(wid * CPW, CPW)],
                    dstv.at[pl.ds(0, CPW)])
    pltpu.sync_copy(ones_hbm, onesv)

    @pl.when(wid < XW)
    def _():
        pltpu.sync_copy(dst_hbm.at[pl.ds(NW * CPW + wid, 1)],
                        dstv.at[pl.ds(CPW, 1)])

    plsc.subcore_barrier()

    def fire(j, carry):
        pltpu.async_copy(onesv, acc.at[dstv.at[j]], sem, add=True)
        return carry

    lax.fori_loop(0, CPW, fire, 0)

    @pl.when(wid < XW)
    def _():
        pltpu.sync_copy(onesv, acc.at[dstv.at[CPW]], add=True)

    def drain(j, carry):
        pltpu.make_async_copy(onesv, acc.at[dstv.at[0]], sem).wait()
        return carry

    lax.fori_loop(0, CPW, drain, 0)
    plsc.subcore_barrier()
    pltpu.sync_copy(acc.at[pl.ds(s * RPT, RPT)],
                    deg_out.at[c, pl.ds(s * RPT, RPT)])


def _edge_body(hp_hbm, src_hbm, dst_hbm, zeros_hbm, acc_out,
               srcv, dstv, rows_a, rows_b, acc, sem_a, sem_b):
    c = lax.axis_index("c")
    s = lax.axis_index("s")
    wid = s * NC + c

    @pl.when(c == 0)
    def _():
        pltpu.sync_copy(hp_hbm.at[pl.ds(s * RPT, RPT)],
                        acc.at[pl.ds(s * RPT, RPT)])

    @pl.when(c != 0)
    def _():
        pltpu.sync_copy(zeros_hbm, acc.at[pl.ds(s * RPT, RPT)])

    pltpu.sync_copy(src_hbm.at[pl.ds(wid * CPW, CPW)],
                    srcv.at[pl.ds(0, CPW)])
    pltpu.sync_copy(dst_hbm.at[pl.ds(wid * CPW, CPW)],
                    dstv.at[pl.ds(0, CPW)])

    @pl.when(wid < XW)
    def _():
        pltpu.sync_copy(src_hbm.at[pl.ds(NW * CPW + wid, 1)],
                        srcv.at[pl.ds(CPW, 1)])
        pltpu.sync_copy(dst_hbm.at[pl.ds(NW * CPW + wid, 1)],
                        dstv.at[pl.ds(CPW, 1)])

    plsc.subcore_barrier()

    pltpu.async_copy(hp_hbm.at[srcv.at[0]], rows_a, sem_a)

    def body(j2, carry):
        j = j2 * 2
        pltpu.make_async_copy(hp_hbm.at[srcv.at[j]], rows_a, sem_a).wait()
        pltpu.async_copy(hp_hbm.at[srcv.at[j + 1]], rows_b, sem_b)
        pltpu.sync_copy(rows_a, acc.at[dstv.at[j]], add=True)
        pltpu.make_async_copy(hp_hbm.at[srcv.at[j + 1]], rows_b, sem_b).wait()
        nxt = jnp.minimum(j + 2, CPW - 1)
        pltpu.async_copy(hp_hbm.at[srcv.at[nxt]], rows_a, sem_a)
        pltpu.sync_copy(rows_b, acc.at[dstv.at[j + 1]], add=True)
        return carry

    lax.fori_loop(0, CPW // 2, body, 0)
    pltpu.make_async_copy(hp_hbm.at[srcv.at[CPW - 1]], rows_a, sem_a).wait()

    @pl.when(wid < XW)
    def _():
        pltpu.async_copy(hp_hbm.at[srcv.at[CPW]], rows_a, sem_a).wait()
        pltpu.sync_copy(rows_a, acc.at[dstv.at[CPW]], add=True)

    plsc.subcore_barrier()
    pltpu.sync_copy(acc.at[pl.ds(s * RPT, RPT)],
                    acc_out.at[c, pl.ds(s * RPT, RPT)])


_BLK = 1000


def _scale_mm_body(x_ref, wt_ref, deg_ref, hp_ref):
    deg = deg_ref[0, :, 0:1] + deg_ref[1, :, 0:1] + 1.0
    dis = jnp.broadcast_to(lax.rsqrt(deg), (_BLK, DP))
    h = jnp.dot(x_ref[...], wt_ref[...], preferred_element_type=jnp.float32)
    hp_ref[...] = dis * h


_scale_mm = pl.pallas_call(
    _scale_mm_body,
    grid=(N // _BLK,),
    in_specs=[
        pl.BlockSpec((_BLK, IN_DIM), lambda i: (i, 0)),
        pl.BlockSpec((IN_DIM, DP), lambda i: (0, 0)),
        pl.BlockSpec((NC, _BLK, 16), lambda i: (0, i, 0)),
    ],
    out_specs=pl.BlockSpec((_BLK, DP), lambda i: (i, 0)),
    out_shape=jax.ShapeDtypeStruct((N, DP), jnp.float32),
)


def _epilogue_body(acc_ref, deg_ref, b_ref, out_ref):
    deg = deg_ref[0, :, 0:1] + deg_ref[1, :, 0:1] + 1.0
    dis = lax.rsqrt(deg)
    agg = acc_ref[0] + acc_ref[1]
    out_ref[...] = (dis * agg)[:, :OUT_DIM] + b_ref[0:1, :OUT_DIM]


_epilogue = pl.pallas_call(
    _epilogue_body,
    grid=(N // _BLK,),
    in_specs=[
        pl.BlockSpec((NC, _BLK, DP), lambda i: (0, i, 0)),
        pl.BlockSpec((NC, _BLK, 16), lambda i: (0, i, 0)),
        pl.BlockSpec((8, DP), lambda i: (0, 0)),
    ],
    out_specs=pl.BlockSpec((_BLK, OUT_DIM), lambda i: (i, 0)),
    out_shape=jax.ShapeDtypeStruct((N, OUT_DIM), jnp.float32),
)


@jax.jit
def kernel(x, edge_index, W, b):
    src2 = edge_index[0].astype(jnp.int32).reshape(ROWS, CW)
    dst2 = edge_index[1].astype(jnp.int32).reshape(ROWS, CW)

    wt_p = jnp.pad(W.T, ((0, 0), (0, DP - OUT_DIM)))
    b_p = jnp.broadcast_to(jnp.pad(b, (0, DP - OUT_DIM))[None, :], (8, DP))

    zeros16 = jnp.zeros((RPT, 16), jnp.float32)
    ones16 = jnp.ones((CW, 16), jnp.float32)
    zerosDP = jnp.zeros((RPT, DP), jnp.float32)

    deg_kernel, edge_kernel = _sc_kernels()
    deg2 = deg_kernel(dst2, zeros16, ones16)
    hp = _scale_mm(x, wt_p, deg2)
    acc2 = edge_kernel(hp, src2, dst2, zerosDP)
    return _epilogue(acc2, deg2, b_p)


# speedup vs baseline: 3.3562x; 1.0966x over previous
import functools
import jax
import jax.numpy as jnp
from jax import lax
from jax.experimental import pallas as pl
from jax.experimental.pallas import tpu as pltpu
from jax.experimental.pallas import tpu_sc as plsc

N = 10000
E = 320000
IN_DIM = 128
OUT_DIM = 100

DP = 112
NC = 2
NS = 16
NW = NC * NS
CW = 128
ROWS = E // CW
CPW = ROWS // NW
XW = ROWS - NW * CPW
RPT = N // NS
HR = CPW // 2


@functools.cache
def _sc_kernels():
    mesh = plsc.VectorSubcoreMesh(
        core_axis_name="c", subcore_axis_name="s",
        num_cores=NC, num_subcores=NS)

    params = pltpu.CompilerParams(use_tc_tiling_on_sc=False)

    deg_kernel = functools.partial(
        pl.kernel,
        out_type=jax.ShapeDtypeStruct((NC, N, 16), jnp.float32),
        mesh=mesh,
        compiler_params=params,
        scratch_types=[
            pltpu.VMEM((CPW + 1, CW), jnp.int32),
            pltpu.VMEM((CW, 16), jnp.float32),
            pltpu.VMEM_SHARED((N, 16), jnp.float32),
            pltpu.SemaphoreType.DMA,
        ],
    )(_deg_body)

    edge_kernel = functools.partial(
        pl.kernel,
        out_type=jax.ShapeDtypeStruct((NC, N, DP), jnp.float32),
        mesh=mesh,
        compiler_params=params,
        scratch_types=[
            pltpu.VMEM((HR + 1, CW), jnp.int32),
            pltpu.VMEM((HR + 1, CW), jnp.int32),
            pltpu.VMEM((CW, DP), jnp.float32),
            pltpu.VMEM((CW, DP), jnp.float32),
            pltpu.VMEM((CW, DP), jnp.float32),
            pltpu.VMEM_SHARED((N, DP), jnp.float32),
        ] + [pltpu.SemaphoreType.DMA] * 6,
    )(_edge_body)

    return deg_kernel, edge_kernel


def _deg_body(dst_hbm, zeros_hbm, ones_hbm, deg_out, dstv, onesv, acc, sem):
    c = lax.axis_index("c")
    s = lax.axis_index("s")
    wid = s * NC + c
    pltpu.sync_copy(zeros_hbm, acc.at[pl.ds(s * RPT, RPT)])
    pltpu.sync_copy(dst_hbm.at[pl.ds(wid * CPW, CPW)],
                    dstv.at[pl.ds(0, CPW)])
    pltpu.sync_copy(ones_hbm, onesv)

    @pl.when(wid < XW)
    def _():
        pltpu.sync_copy(dst_hbm.at[pl.ds(NW * CPW + wid, 1)],
                        dstv.at[pl.ds(CPW, 1)])

    plsc.subcore_barrier()

    def fire(j, carry):
        pltpu.async_copy(onesv, acc.at[dstv.at[j]], sem, add=True)
        return carry

    lax.fori_loop(0, CPW, fire, 0)

    @pl.when(wid < XW)
    def _():
        pltpu.sync_copy(onesv, acc.at[dstv.at[CPW]], add=True)

    def drain(j, carry):
        pltpu.make_async_copy(onesv, acc.at[dstv.at[0]], sem).wait()
        return carry

    lax.fori_loop(0, CPW, drain, 0)
    plsc.subcore_barrier()
    pltpu.sync_copy(acc.at[pl.ds(s * RPT, RPT)],
                    deg_out.at[c, pl.ds(s * RPT, RPT)])


def _edge_body(hp_hbm, src_hbm, dst_hbm, zeros_hbm, acc_out,
               srcv, dstv, r0, r1, r2, acc, g0, g1, g2, s0, s1, s2):
    rows = (r0, r1, r2)
    gsem = (g0, g1, g2)
    ssem = (s0, s1, s2)
    c = lax.axis_index("c")
    s = lax.axis_index("s")
    wid = s * NC + c

    @pl.when(c == 0)
    def _():
        pltpu.sync_copy(hp_hbm.at[pl.ds(s * RPT, RPT)],
                        acc.at[pl.ds(s * RPT, RPT)])

    @pl.when(c != 0)
    def _():
        pltpu.sync_copy(zeros_hbm, acc.at[pl.ds(s * RPT, RPT)])

    plsc.subcore_barrier()
    pltpu.sync_copy(zeros_hbm.at[pl.ds(0, CW)], rows[2])

    for h in range(2):
        pltpu.sync_copy(src_hbm.at[pl.ds(wid * CPW + h * HR, HR)],
                        srcv.at[pl.ds(0, HR)])
        pltpu.sync_copy(dst_hbm.at[pl.ds(wid * CPW + h * HR, HR)],
                        dstv.at[pl.ds(0, HR)])
        pltpu.async_copy(hp_hbm.at[srcv.at[0]], rows[0], gsem[0])
        pltpu.async_copy(hp_hbm.at[srcv.at[1]], rows[1], gsem[1])
        pltpu.async_copy(rows[2], acc.at[dstv.at[0]], ssem[2], add=True)

        def body(j3, carry):
            for k in range(3):
                m = j3 * 3 + k
                pltpu.make_async_copy(
                    hp_hbm.at[srcv.at[m]], rows[k], gsem[k]).wait()
                pltpu.async_copy(rows[k], acc.at[dstv.at[m]], ssem[k],
                                 add=True)
                kn = (k + 2) % 3
                pltpu.make_async_copy(
                    rows[kn], acc.at[dstv.at[0]], ssem[kn]).wait()
                nxt = jnp.minimum(m + 2, HR - 1)
                pltpu.async_copy(hp_hbm.at[srcv.at[nxt]], rows[kn], gsem[kn])
            return carry

        lax.fori_loop(0, HR // 3, body, 0)
        pltpu.make_async_copy(
            hp_hbm.at[srcv.at[HR - 1]], rows[0], gsem[0]).wait()
        pltpu.make_async_copy(
            hp_hbm.at[srcv.at[HR - 1]], rows[1], gsem[1]).wait()
        pltpu.make_async_copy(rows[2], acc.at[dstv.at[0]], ssem[2]).wait()
        if h == 0:
            pltpu.sync_copy(zeros_hbm.at[pl.ds(0, CW)], rows[2])

    @pl.when(wid < XW)
    def _():
        pltpu.sync_copy(src_hbm.at[pl.ds(NW * CPW + wid, 1)],
                        srcv.at[pl.ds(HR, 1)])
        pltpu.sync_copy(dst_hbm.at[pl.ds(NW * CPW + wid, 1)],
                        dstv.at[pl.ds(HR, 1)])
        pltpu.async_copy(hp_hbm.at[srcv.at[HR]], rows[0], gsem[0]).wait()
        pltpu.sync_copy(rows[0], acc.at[dstv.at[HR]], add=True)

    plsc.subcore_barrier()
    pltpu.sync_copy(acc.at[pl.ds(s * RPT, RPT)],
                    acc_out.at[c, pl.ds(s * RPT, RPT)])


_BLK = 1000


def _scale_mm_body(x_ref, wt_ref, deg_ref, hp_ref):
    deg = deg_ref[0, :, 0:1] + deg_ref[1, :, 0:1] + 1.0
    dis = jnp.broadcast_to(lax.rsqrt(deg), (_BLK, DP))
    h = jnp.dot(x_ref[...], wt_ref[...], preferred_element_type=jnp.float32)
    hp_ref[...] = dis * h


_scale_mm = pl.pallas_call(
    _scale_mm_body,
    grid=(N // _BLK,),
    in_specs=[
        pl.BlockSpec((_BLK, IN_DIM), lambda i: (i, 0)),
        pl.BlockSpec((IN_DIM, DP), lambda i: (0, 0)),
        pl.BlockSpec((NC, _BLK, 16), lambda i: (0, i, 0)),
    ],
    out_specs=pl.BlockSpec((_BLK, DP), lambda i: (i, 0)),
    out_shape=jax.ShapeDtypeStruct((N, DP), jnp.float32),
)


def _epilogue_body(acc_ref, deg_ref, b_ref, out_ref):
    deg = deg_ref[0, :, 0:1] + deg_ref[1, :, 0:1] + 1.0
    dis = lax.rsqrt(deg)
    agg = acc_ref[0] + acc_ref[1]
    out_ref[...] = (dis * agg)[:, :OUT_DIM] + b_ref[0:1, :OUT_DIM]


_epilogue = pl.pallas_call(
    _epilogue_body,
    grid=(N // _BLK,),
    in_specs=[
        pl.BlockSpec((NC, _BLK, DP), lambda i: (0, i, 0)),
        pl.BlockSpec((NC, _BLK, 16), lambda i: (0, i, 0)),
        pl.BlockSpec((8, DP), lambda i: (0, 0)),
    ],
    out_specs=pl.BlockSpec((_BLK, OUT_DIM), lambda i: (i, 0)),
    out_shape=jax.ShapeDtypeStruct((N, OUT_DIM), jnp.float32),
)


@jax.jit
def kernel(x, edge_index, W, b):
    src2 = edge_index[0].astype(jnp.int32).reshape(ROWS, CW)
    dst2 = edge_index[1].astype(jnp.int32).reshape(ROWS, CW)

    wt_p = jnp.pad(W.T, ((0, 0), (0, DP - OUT_DIM)))
    b_p = jnp.broadcast_to(jnp.pad(b, (0, DP - OUT_DIM))[None, :], (8, DP))

    zeros16 = jnp.zeros((RPT, 16), jnp.float32)
    ones16 = jnp.ones((CW, 16), jnp.float32)
    zerosDP = jnp.zeros((RPT, DP), jnp.float32)

    deg_kernel, edge_kernel = _sc_kernels()
    deg2 = deg_kernel(dst2, zeros16, ones16)
    hp = _scale_mm(x, wt_p, deg2)
    acc2 = edge_kernel(hp, src2, dst2, zerosDP)
    return _epilogue(acc2, deg2, b_p)
